# baseline (speedup 1.0000x reference)
import jax
import jax.numpy as jnp
from jax.experimental import pallas as pl
from jax.experimental.pallas import tpu as pltpu


def _scale_body(scale_ref, x_ref, o_ref):
    b = pl.program_id(0)
    o_ref[...] = x_ref[...] * scale_ref[b]


def kernel(input, probs, magnitudes):
    B, C, H, W = input.shape
    scale = jnp.where(probs, magnitudes, jnp.float32(1.0))
    M = (C * H * W) // 128
    x = input.reshape(B, M, 128)
    MB = M // 4
    grid = (B, M // MB)
    out = pl.pallas_call(
        _scale_body,
        grid_spec=pltpu.PrefetchScalarGridSpec(
            num_scalar_prefetch=1,
            grid=grid,
            in_specs=[pl.BlockSpec((1, MB, 128), lambda b, m, s: (b, m, 0))],
            out_specs=pl.BlockSpec((1, MB, 128), lambda b, m, s: (b, m, 0)),
        ),
        out_shape=jax.ShapeDtypeStruct((B, M, 128), jnp.float32),
    )(scale, x)
    return out.reshape(B, C, H, W)


# speedup vs baseline: 1.0523x; 1.0523x over previous
import jax
import jax.numpy as jnp
from jax.experimental import pallas as pl
from jax.experimental.pallas import tpu as pltpu


def _scale_body(scale_ref, x_ref, o_ref):
    b = pl.program_id(0)
    o_ref[...] = x_ref[...] * scale_ref[b]


def kernel(input, probs, magnitudes):
    B, C, H, W = input.shape
    scale = jnp.where(probs, magnitudes, jnp.float32(1.0))
    LANES = 16384
    ROWS = (C * H * W) // LANES
    x = input.reshape(B * ROWS, LANES)
    grid = (B,)
    out = pl.pallas_call(
        _scale_body,
        grid_spec=pltpu.PrefetchScalarGridSpec(
            num_scalar_prefetch=1,
            grid=grid,
            in_specs=[pl.BlockSpec((ROWS, LANES), lambda b, s: (b, 0))],
            out_specs=pl.BlockSpec((ROWS, LANES), lambda b, s: (b, 0)),
        ),
        out_shape=jax.ShapeDtypeStruct((B * ROWS, LANES), jnp.float32),
    )(scale, x)
    return out.reshape(B, C, H, W)


# speedup vs baseline: 4.7339x; 4.4987x over previous
import jax
import jax.numpy as jnp
from jax.experimental import pallas as pl
from jax.experimental.pallas import tpu as pltpu


def _scale_body(scale_ref, x_ref, o_ref):
    b = pl.program_id(0)
    o_ref[...] = x_ref[...] * scale_ref[b]


def kernel(input, probs, magnitudes):
    B, C, H, W = input.shape
    scale = jnp.where(probs, magnitudes, jnp.float32(1.0))
    grid = (B,)
    out = pl.pallas_call(
        _scale_body,
        grid_spec=pltpu.PrefetchScalarGridSpec(
            num_scalar_prefetch=1,
            grid=grid,
            in_specs=[pl.BlockSpec((1, C, H, W), lambda b, s: (b, 0, 0, 0))],
            out_specs=pl.BlockSpec((1, C, H, W), lambda b, s: (b, 0, 0, 0)),
        ),
        out_shape=jax.ShapeDtypeStruct((B, C, H, W), jnp.float32),
    )(scale, input)
    return out
